# Initial kernel scaffold; baseline (speedup 1.0000x reference)
import jax
import jax.numpy as jnp
from jax.experimental import pallas as pl


def kernel(features, edge_index, edge_weight, W1, b1, W2, b2):
    raise NotImplementedError("write your pallas kernel here")


# speedup vs baseline: 11.2957x; 11.2957x over previous
import functools

import jax
import jax.numpy as jnp
from jax import lax
from jax.experimental import pallas as pl
from jax.experimental.pallas import tpu as pltpu
from jax.experimental.pallas import tpu_sc as plsc

NC = 2
NS = 16
NW = NC * NS
K = 80


def _sc_mesh():
    return plsc.VectorSubcoreMesh(core_axis_name="c", subcore_axis_name="s")


_SC_PARAMS = pltpu.CompilerParams(use_tc_tiling_on_sc=False)


def _make_deg(NP, E):
    nch = (E // NW) // K
    CZ = NP // NS

    @functools.partial(
        pl.kernel,
        out_type=jax.ShapeDtypeStruct((NC, 1, NP), jnp.float32),
        mesh=_sc_mesh(),
        compiler_params=_SC_PARAMS,
        scratch_types=[
            pltpu.VMEM((nch, K), jnp.int32),
            pltpu.VMEM((nch, K), jnp.float32),
            pltpu.VMEM_SHARED((NP,), jnp.float32),
            pltpu.SemaphoreType.DMA,
        ],
    )
    def deg_kernel(col_hbm, w_hbm, z_hbm, out_hbm, colv, wv, dacc, sem):
        cid = lax.axis_index("c")
        sid = lax.axis_index("s")
        wid = sid * NC + cid

        pltpu.sync_copy(z_hbm, dacc.at[pl.ds(sid * CZ, CZ)])
        plsc.subcore_barrier()
        pltpu.sync_copy(col_hbm.at[wid], colv)
        pltpu.sync_copy(w_hbm.at[wid], wv)

        def body(j, carry):
            pltpu.sync_copy(wv.at[j], dacc.at[colv.at[j]], add=True)
            return carry

        lax.fori_loop(0, nch, body, 0)
        plsc.subcore_barrier()
        pltpu.sync_copy(dacc.at[pl.ds(sid * CZ, CZ)],
                        out_hbm.at[cid, 0, pl.ds(sid * CZ, CZ)])

    return deg_kernel


def _scale_and_scatter(msg, wv, colv, acc, K_, NQ):
    def scale_grp(t, c2):
        wvec = wv[pl.ds(t * 16, 16)]
        base = t * 16
        for u in range(16):
            ws = wvec[u]
            for q in range(NQ):
                msg[base + u, pl.ds(q * 16, 16)] = (
                    msg[base + u, pl.ds(q * 16, 16)] * ws)
        return c2

    lax.fori_loop(0, K_ // 16, scale_grp, 0)
    pltpu.sync_copy(msg, acc.at[colv], add=True)


def _make_agg_split(NP, E, D):
    Dh = D // NC
    nch = (E // NS) // K
    RPS = NP // NS
    NQ = Dh // 16

    @functools.partial(
        pl.kernel,
        out_type=jax.ShapeDtypeStruct((NC, NP, Dh), jnp.float32),
        mesh=_sc_mesh(),
        compiler_params=_SC_PARAMS,
        scratch_types=[
            pltpu.VMEM((nch, K), jnp.int32),
            pltpu.VMEM((nch, K), jnp.int32),
            pltpu.VMEM((nch, K), jnp.float32),
            pltpu.VMEM((K, Dh), jnp.float32),
            pltpu.VMEM_SHARED((NP, Dh), jnp.float32),
            pltpu.SemaphoreType.DMA,
        ],
    )
    def agg_kernel(g0_hbm, g1_hbm, row_hbm, col_hbm, w_hbm, z_hbm, out_hbm,
                   rowv, colv, wv, msg, acc, sem):
        cid = lax.axis_index("c")
        sid = lax.axis_index("s")

        pltpu.sync_copy(z_hbm, acc.at[pl.ds(sid * RPS, RPS)])
        plsc.subcore_barrier()

        pltpu.sync_copy(row_hbm.at[sid], rowv)
        pltpu.sync_copy(col_hbm.at[sid], colv)
        pltpu.sync_copy(w_hbm.at[sid], wv)

        def chunk(j, carry):
            @pl.when(cid == 0)
            def _g0():
                pltpu.async_copy(g0_hbm.at[rowv.at[j]], msg, sem).wait()

            @pl.when(cid == 1)
            def _g1():
                pltpu.async_copy(g1_hbm.at[rowv.at[j]], msg, sem).wait()

            _scale_and_scatter(msg, wv.at[j], colv.at[j], acc, K, NQ)
            return carry

        lax.fori_loop(0, nch, chunk, 0)
        plsc.subcore_barrier()
        pltpu.sync_copy(acc.at[pl.ds(sid * RPS, RPS)],
                        out_hbm.at[cid, pl.ds(sid * RPS, RPS)])

    return agg_kernel


def _make_agg(NP, E, D):
    nch = (E // NW) // K
    RPS = NP // NS
    NQ = D // 16

    @functools.partial(
        pl.kernel,
        out_type=jax.ShapeDtypeStruct((NC, NP, D), jnp.float32),
        mesh=_sc_mesh(),
        compiler_params=_SC_PARAMS,
        scratch_types=[
            pltpu.VMEM((nch, K), jnp.int32),
            pltpu.VMEM((nch, K), jnp.int32),
            pltpu.VMEM((nch, K), jnp.float32),
            pltpu.VMEM((K, D), jnp.float32),
            pltpu.VMEM_SHARED((NP, D), jnp.float32),
            pltpu.SemaphoreType.DMA,
        ],
    )
    def agg_kernel(g_hbm, row_hbm, col_hbm, w_hbm, z_hbm, out_hbm,
                   rowv, colv, wv, msg, acc, sem):
        cid = lax.axis_index("c")
        sid = lax.axis_index("s")
        wid = sid * NC + cid

        pltpu.sync_copy(z_hbm, acc.at[pl.ds(sid * RPS, RPS)])
        plsc.subcore_barrier()

        pltpu.sync_copy(row_hbm.at[wid], rowv)
        pltpu.sync_copy(col_hbm.at[wid], colv)
        pltpu.sync_copy(w_hbm.at[wid], wv)

        def chunk(j, carry):
            pltpu.async_copy(g_hbm.at[rowv.at[j]], msg, sem).wait()
            _scale_and_scatter(msg, wv.at[j], colv.at[j], acc, K, NQ)
            return carry

        lax.fori_loop(0, nch, chunk, 0)
        plsc.subcore_barrier()
        pltpu.sync_copy(acc.at[pl.ds(sid * RPS, RPS)],
                        out_hbm.at[cid, pl.ds(sid * RPS, RPS)])

    return agg_kernel


def _tc1(x, W1, degp_t, BN=1280):
    NP, F = x.shape
    H = W1.shape[1]

    def body(x_ref, w_ref, dp_ref, h_ref, g_ref, dinv_ref):
        dp = dp_ref[...]
        deg = dp[:, 0:1] + dp[:, 1:2] + 1.0
        dinv = jnp.where(deg > 0, lax.rsqrt(deg), 0.0)
        h = jnp.dot(x_ref[...], w_ref[...], preferred_element_type=jnp.float32)
        h_ref[...] = h
        g_ref[...] = h * dinv
        dinv_ref[...] = dinv

    return pl.pallas_call(
        body,
        grid=(NP // BN,),
        in_specs=[
            pl.BlockSpec((BN, F), lambda i: (i, 0)),
            pl.BlockSpec((F, H), lambda i: (0, 0)),
            pl.BlockSpec((BN, NC), lambda i: (i, 0)),
        ],
        out_specs=[
            pl.BlockSpec((BN, H), lambda i: (i, 0)),
            pl.BlockSpec((BN, H), lambda i: (i, 0)),
            pl.BlockSpec((BN, 1), lambda i: (i, 0)),
        ],
        out_shape=[
            jax.ShapeDtypeStruct((NP, H), jnp.float32),
            jax.ShapeDtypeStruct((NP, H), jnp.float32),
            jax.ShapeDtypeStruct((NP, 1), jnp.float32),
        ],
    )(x, W1, degp_t)


def _tc2(agg1, h1, dinv2d, W2p, b1r, BN=1280):
    NP, H = h1.shape
    Dp = W2p.shape[1]

    def body(a_ref, h1_ref, dinv_ref, w2_ref, b1_ref, h2_ref, g2_ref):
        dinv = dinv_ref[...]
        out1 = jnp.maximum(
            dinv * a_ref[...] + (dinv * dinv) * h1_ref[...] + b1_ref[...], 0.0)
        h2 = jnp.dot(out1, w2_ref[...], preferred_element_type=jnp.float32)
        h2_ref[...] = h2
        g2_ref[...] = dinv * h2

    return pl.pallas_call(
        body,
        grid=(NP // BN,),
        in_specs=[
            pl.BlockSpec((BN, H), lambda i: (i, 0)),
            pl.BlockSpec((BN, H), lambda i: (i, 0)),
            pl.BlockSpec((BN, 1), lambda i: (i, 0)),
            pl.BlockSpec((H, Dp), lambda i: (0, 0)),
            pl.BlockSpec((1, H), lambda i: (0, 0)),
        ],
        out_specs=[
            pl.BlockSpec((BN, Dp), lambda i: (i, 0)),
            pl.BlockSpec((BN, Dp), lambda i: (i, 0)),
        ],
        out_shape=[
            jax.ShapeDtypeStruct((NP, Dp), jnp.float32),
            jax.ShapeDtypeStruct((NP, Dp), jnp.float32),
        ],
    )(agg1, h1, dinv2d, W2p, b1r)


def _tc3(parts2, h2, dinv2d, b2r, C, BN=1280):
    NP, Dp = h2.shape

    def body(p_ref, h2_ref, dinv_ref, b2_ref, out_ref):
        dinv = dinv_ref[...]
        p = p_ref[...]
        x = dinv * (p[0] + p[1]) + (dinv * dinv) * h2_ref[...] + b2_ref[...]
        valid = lax.broadcasted_iota(jnp.int32, x.shape, 1) < C
        x = jnp.where(valid, x, -1e30)
        m = jnp.max(x, axis=1, keepdims=True)
        ls = jnp.log(jnp.sum(jnp.exp(x - m), axis=1, keepdims=True))
        out_ref[...] = x - m - ls

    return pl.pallas_call(
        body,
        grid=(NP // BN,),
        in_specs=[
            pl.BlockSpec((NC, BN, Dp), lambda i: (0, i, 0)),
            pl.BlockSpec((BN, Dp), lambda i: (i, 0)),
            pl.BlockSpec((BN, 1), lambda i: (i, 0)),
            pl.BlockSpec((1, Dp), lambda i: (0, 0)),
        ],
        out_specs=pl.BlockSpec((BN, Dp), lambda i: (i, 0)),
        out_shape=jax.ShapeDtypeStruct((NP, Dp), jnp.float32),
    )(parts2, h2, dinv2d, b2r)


def kernel(features, edge_index, edge_weight, W1, b1, W2, b2):
    N, F = features.shape
    H = W1.shape[1]
    C = W2.shape[1]
    E = edge_weight.shape[0]
    Dp = ((C + 15) // 16) * 16
    NP = ((N + 2047) // 2048) * 2048
    nch = (E // NW) // K

    x_p = jnp.pad(features, ((0, NP - N), (0, 0)))
    row3d = edge_index[0].reshape(NW, nch, K)
    col3d = edge_index[1].reshape(NW, nch, K)
    w3d = edge_weight.reshape(NW, nch, K)
    row3s = edge_index[0].reshape(NS, nch * NC, K)
    col3s = edge_index[1].reshape(NS, nch * NC, K)
    w3s = edge_weight.reshape(NS, nch * NC, K)

    zdeg = jnp.zeros((NP // NS,), jnp.float32)
    zH = jnp.zeros((NP // NS, H // NC), jnp.float32)
    zD = jnp.zeros((NP // NS, Dp), jnp.float32)

    W2p = jnp.pad(W2, ((0, 0), (0, Dp - C)))
    b1r = b1.reshape(1, H)
    b2r = jnp.pad(b2, (0, Dp - C)).reshape(1, Dp)

    degp = _make_deg(NP, E)(col3d, w3d, zdeg)
    degp_t = degp.reshape(NC, NP).T
    h1, g1, dinv2d = _tc1(x_p, W1, degp_t)
    g1a = g1[:, : H // NC]
    g1b = g1[:, H // NC:]
    parts1 = _make_agg_split(NP, E, H)(
        g1a, g1b, row3s, col3s, w3s, zH)
    agg1 = jnp.concatenate([parts1[0], parts1[1]], axis=1)
    h2, g2 = _tc2(agg1, h1, dinv2d, W2p, b1r)
    parts2 = _make_agg(NP, E, Dp)(g2, row3d, col3d, w3d, zD)
    out = _tc3(parts2, h2, dinv2d, b2r, C)
    return out[:N, :C]
